# Initial kernel scaffold; baseline (speedup 1.0000x reference)
import jax
import jax.numpy as jnp
from jax.experimental import pallas as pl


def kernel(h, A, W):
    raise NotImplementedError("write your pallas kernel here")


# speedup vs baseline: 1.7232x; 1.7232x over previous
import jax
import jax.numpy as jnp
from jax.experimental import pallas as pl
from jax.experimental.pallas import tpu as pltpu


def _agg_body(a_ref, h_ref, u_ref, s_ref):
    i = pl.program_id(0)

    @pl.when(i == 0)
    def _init_s():
        s_ref[...] = jnp.zeros_like(s_ref)

    a = a_ref[...]
    u_ref[...] = jnp.dot(a, h_ref[...], preferred_element_type=jnp.float32)
    s_ref[...] += jnp.sum(a)[None, None]


def _proj_body(h_ref, u_ref, wa_ref, wb_ref, s_ref, o_ref):
    inv = 1.0 / s_ref[0, 0]
    o = jnp.dot(h_ref[...], wa_ref[...], preferred_element_type=jnp.float32)
    o += jnp.dot(u_ref[...], wb_ref[...], preferred_element_type=jnp.float32) * inv
    o_ref[...] = jnp.maximum(o, 0.0)


def kernel(h, A, W):
    n, d = h.shape
    bi = 400

    u, s = pl.pallas_call(
        _agg_body,
        grid=(n // bi,),
        in_specs=[
            pl.BlockSpec((bi, n), lambda i: (i, 0)),
            pl.BlockSpec((n, d), lambda i: (0, 0)),
        ],
        out_specs=[
            pl.BlockSpec((bi, d), lambda i: (i, 0)),
            pl.BlockSpec((1, 1), lambda i: (0, 0)),
        ],
        out_shape=[
            jax.ShapeDtypeStruct((n, d), jnp.float32),
            jax.ShapeDtypeStruct((1, 1), jnp.float32),
        ],
        compiler_params=pltpu.CompilerParams(
            dimension_semantics=("arbitrary",),
        ),
    )(A, h)

    wt = W.T
    wa = wt[:d]
    wb = wt[d:]

    be = 2000
    out = pl.pallas_call(
        _proj_body,
        grid=(n // be,),
        in_specs=[
            pl.BlockSpec((be, d), lambda i: (i, 0)),
            pl.BlockSpec((be, d), lambda i: (i, 0)),
            pl.BlockSpec((d, d), lambda i: (0, 0)),
            pl.BlockSpec((d, d), lambda i: (0, 0)),
            pl.BlockSpec((1, 1), lambda i: (0, 0)),
        ],
        out_specs=pl.BlockSpec((be, d), lambda i: (i, 0)),
        out_shape=jax.ShapeDtypeStruct((n, d), jnp.float32),
    )(h, u, wa, wb, s)
    return out
